# Initial kernel scaffold; baseline (speedup 1.0000x reference)
import jax
import jax.numpy as jnp
from jax.experimental import pallas as pl


def kernel(embeddings, labels, W):
    raise NotImplementedError("write your pallas kernel here")


# speedup vs baseline: 3.8810x; 3.8810x over previous
import jax
import jax.numpy as jnp
from jax.experimental import pallas as pl


_BATCH = 4096
_CLASSES = 1000
_DIM = 64
_BLK = 512
_INV_TEMP = 20.0


def _loss_kernel(e_ref, w_ref, lab_ref, out_ref):
    i = pl.program_id(0)
    e = e_ref[...]
    w = w_ref[...]
    en = e / (jnp.sqrt(jnp.sum(e * e, axis=1, keepdims=True)) + 1e-12)
    wn = w / (jnp.sqrt(jnp.sum(w * w, axis=1, keepdims=True)) + 1e-12)
    logits = jax.lax.dot_general(
        en, wn, (((1,), (1,)), ((), ())), preferred_element_type=jnp.float32
    ) * _INV_TEMP
    m = jnp.max(logits, axis=1, keepdims=True)
    lse = m[:, 0] + jnp.log(jnp.sum(jnp.exp(logits - m), axis=1))
    labs = lab_ref[0, 0, :]
    col = jax.lax.broadcasted_iota(jnp.int32, (_BLK, _CLASSES), 1)
    tgt = jnp.sum(jnp.where(col == labs[:, None], logits, 0.0), axis=1)
    part = jnp.sum(lse - tgt).reshape(1, 1)

    @pl.when(i == 0)
    def _():
        out_ref[...] = jnp.zeros((1, 1), jnp.float32)

    out_ref[...] += part


def kernel(embeddings, labels, W):
    labs3 = labels.astype(jnp.int32).reshape(_BATCH // _BLK, 1, _BLK)
    loss_sum = pl.pallas_call(
        _loss_kernel,
        grid=(_BATCH // _BLK,),
        in_specs=[
            pl.BlockSpec((_BLK, _DIM), lambda i: (i, 0)),
            pl.BlockSpec((_CLASSES, _DIM), lambda i: (0, 0)),
            pl.BlockSpec((1, 1, _BLK), lambda i: (i, 0, 0)),
        ],
        out_specs=pl.BlockSpec((1, 1), lambda i: (0, 0)),
        out_shape=jax.ShapeDtypeStruct((1, 1), jnp.float32),
    )(embeddings, W, labs3)
    loss = loss_sum[0, 0] / _BATCH
    combined_labels = jnp.concatenate([labels, labels], axis=0)
    return (loss, combined_labels)


# speedup vs baseline: 4.1422x; 1.0673x over previous
import jax
import jax.numpy as jnp
from jax.experimental import pallas as pl


_BATCH = 4096
_CLASSES = 1000
_DIM = 64
_BLK = 512
_INV_TEMP = 20.0


def _loss_kernel(e_ref, w_ref, lab_ref, out_ref):
    i = pl.program_id(0)
    e = e_ref[...]
    w = w_ref[...]
    en = e / (jnp.sqrt(jnp.sum(e * e, axis=1, keepdims=True)) + 1e-12)
    wn = w / (jnp.sqrt(jnp.sum(w * w, axis=1, keepdims=True)) + 1e-12)
    logits = jax.lax.dot_general(
        en, wn, (((1,), (1,)), ((), ())), preferred_element_type=jnp.float32
    ) * _INV_TEMP
    lse = jnp.log(jnp.sum(jnp.exp(logits), axis=1))
    labs = lab_ref[0, 0, :]
    col = jax.lax.broadcasted_iota(jnp.int32, (_BLK, _CLASSES), 1)
    tgt = jnp.sum(jnp.where(col == labs[:, None], logits, 0.0), axis=1)
    part = jnp.sum(lse - tgt).reshape(1, 1)

    @pl.when(i == 0)
    def _():
        out_ref[...] = jnp.zeros((1, 1), jnp.float32)

    out_ref[...] += part


def kernel(embeddings, labels, W):
    labs3 = labels.astype(jnp.int32).reshape(_BATCH // _BLK, 1, _BLK)
    loss_sum = pl.pallas_call(
        _loss_kernel,
        grid=(_BATCH // _BLK,),
        in_specs=[
            pl.BlockSpec((_BLK, _DIM), lambda i: (i, 0)),
            pl.BlockSpec((_CLASSES, _DIM), lambda i: (0, 0)),
            pl.BlockSpec((1, 1, _BLK), lambda i: (i, 0, 0)),
        ],
        out_specs=pl.BlockSpec((1, 1), lambda i: (0, 0)),
        out_shape=jax.ShapeDtypeStruct((1, 1), jnp.float32),
    )(embeddings, W, labs3)
    loss = loss_sum[0, 0] / _BATCH
    combined_labels = jnp.concatenate([labels, labels], axis=0)
    return (loss, combined_labels)


# speedup vs baseline: 4.7135x; 1.1379x over previous
import jax
import jax.numpy as jnp
from jax.experimental import pallas as pl


_BATCH = 4096
_CLASSES = 1000
_DIM = 64
_BLK = 2048
_INV_TEMP = 20.0


def _loss_kernel(e_ref, w_ref, lab_ref, out_ref):
    i = pl.program_id(0)
    e = e_ref[...]
    w = w_ref[...]
    en = e / (jnp.sqrt(jnp.sum(e * e, axis=1, keepdims=True)) + 1e-12)
    wn = w / (jnp.sqrt(jnp.sum(w * w, axis=1, keepdims=True)) + 1e-12)
    logits = jax.lax.dot_general(
        en, wn, (((1,), (1,)), ((), ())), preferred_element_type=jnp.float32
    ) * _INV_TEMP
    lse = jnp.log(jnp.sum(jnp.exp(logits), axis=1))
    labs = lab_ref[0, 0, :]
    col = jax.lax.broadcasted_iota(jnp.int32, (_BLK, _CLASSES), 1)
    tgt = jnp.sum(jnp.where(col == labs[:, None], logits, 0.0), axis=1)
    part = jnp.sum(lse - tgt).reshape(1, 1)

    @pl.when(i == 0)
    def _():
        out_ref[...] = jnp.zeros((1, 1), jnp.float32)

    out_ref[...] += part


def kernel(embeddings, labels, W):
    labs3 = labels.astype(jnp.int32).reshape(_BATCH // _BLK, 1, _BLK)
    loss_sum = pl.pallas_call(
        _loss_kernel,
        grid=(_BATCH // _BLK,),
        in_specs=[
            pl.BlockSpec((_BLK, _DIM), lambda i: (i, 0)),
            pl.BlockSpec((_CLASSES, _DIM), lambda i: (0, 0)),
            pl.BlockSpec((1, 1, _BLK), lambda i: (i, 0, 0)),
        ],
        out_specs=pl.BlockSpec((1, 1), lambda i: (0, 0)),
        out_shape=jax.ShapeDtypeStruct((1, 1), jnp.float32),
    )(embeddings, W, labs3)
    loss = loss_sum[0, 0] / _BATCH
    combined_labels = jnp.concatenate([labels, labels], axis=0)
    return (loss, combined_labels)


# speedup vs baseline: 4.8349x; 1.0258x over previous
import jax
import jax.numpy as jnp
from jax.experimental import pallas as pl


_BATCH = 4096
_CLASSES = 1000
_DIM = 64
_BLK = 4096
_INV_TEMP = 20.0


def _loss_kernel(e_ref, w_ref, lab_ref, out_ref):
    i = pl.program_id(0)
    e = e_ref[...]
    w = w_ref[...]
    en = e / (jnp.sqrt(jnp.sum(e * e, axis=1, keepdims=True)) + 1e-12)
    wn = w / (jnp.sqrt(jnp.sum(w * w, axis=1, keepdims=True)) + 1e-12)
    logits = jax.lax.dot_general(
        en, wn, (((1,), (1,)), ((), ())), preferred_element_type=jnp.float32
    ) * _INV_TEMP
    lse = jnp.log(jnp.sum(jnp.exp(logits), axis=1))
    labs = lab_ref[0, 0, :]
    col = jax.lax.broadcasted_iota(jnp.int32, (_BLK, _CLASSES), 1)
    tgt = jnp.sum(jnp.where(col == labs[:, None], logits, 0.0), axis=1)
    part = jnp.sum(lse - tgt).reshape(1, 1)

    @pl.when(i == 0)
    def _():
        out_ref[...] = jnp.zeros((1, 1), jnp.float32)

    out_ref[...] += part


def kernel(embeddings, labels, W):
    labs3 = labels.astype(jnp.int32).reshape(_BATCH // _BLK, 1, _BLK)
    loss_sum = pl.pallas_call(
        _loss_kernel,
        grid=(_BATCH // _BLK,),
        in_specs=[
            pl.BlockSpec((_BLK, _DIM), lambda i: (i, 0)),
            pl.BlockSpec((_CLASSES, _DIM), lambda i: (0, 0)),
            pl.BlockSpec((1, 1, _BLK), lambda i: (i, 0, 0)),
        ],
        out_specs=pl.BlockSpec((1, 1), lambda i: (0, 0)),
        out_shape=jax.ShapeDtypeStruct((1, 1), jnp.float32),
    )(embeddings, W, labs3)
    loss = loss_sum[0, 0] / _BATCH
    combined_labels = jnp.concatenate([labels, labels], axis=0)
    return (loss, combined_labels)
